# baseline (speedup 1.0000x reference)
import jax
import jax.numpy as jnp
from jax.experimental import pallas as pl
from jax.experimental.pallas import tpu as pltpu
from jax.experimental.pallas import tpu_sc as plsc
import functools

R_MIN = 0.5
R_MAX = 30.0
FLS_H = 512
FLS_W = 512
FOV_H = 130.0
BUFF = 16
P = 256
PS = 8
T = 8
C = 64
DOWN = 4
FH = FLS_H // DOWN
FW = FLS_W // DOWN
SLOT = 2 * P * T
MAX_EDGES = BUFF * SLOT
NPIX = FLS_H * FLS_W
NBLK = 256
KCAND = 512


def _body(xbT_ref, sc3_ref, wcat_ref, fn_ref, ts_ref, time_ref, poses_ref,
          pstate_ref, sframe_ref, i_ref, j_ref, w_ref, v_ref,
          fmap_hbm_in, imap_hbm_in,
          fmap_hbm_out, imap_hbm_out, fmap_out, time_out, poses_out,
          pstate_out, sframe_out, i_out, j_out, w_out, v_out, idx_out,
          cy_out, cx_out,
          imap_s, sem_f, sem_i):
    fn = fn_ref[0]
    local = jax.lax.rem(fn, BUFF)
    ts = ts_ref[0]

    both = jnp.dot(wcat_ref[...], xbT_ref[...],
                   preferred_element_type=jnp.float32)
    fmap_out[...] = both[:C]
    imap_s[...] = jnp.tanh(both[C:])
    cp_f = pltpu.make_async_copy(fmap_out, fmap_hbm_out.at[local], sem_f)
    cp_i = pltpu.make_async_copy(imap_s, imap_hbm_out.at[local], sem_i)
    cp_f.start()
    cp_i.start()

    s2 = sc3_ref[...]
    bits2 = jax.lax.bitcast_convert_type(s2 * s2, jnp.int32)

    def bisect(_, carry):
        lo, hi = carry
        mid = lo + jax.lax.div(hi - lo, 2)
        cnt = jnp.sum((bits2 >= mid).astype(jnp.int32))
        big = cnt >= P
        return (jnp.where(big, mid, lo), jnp.where(big, hi, mid))

    t_bits, _ = jax.lax.fori_loop(0, 31, bisect, (jnp.int32(0),
                                                  jnp.int32(0x7F800000)))

    riota = jax.lax.broadcasted_iota(jnp.int32, (2048, 128), 0)
    liota = jax.lax.broadcasted_iota(jnp.int32, (2048, 128), 1)
    gidx = riota * 128 + liota
    mask_b = bits2 >= t_bits
    mask_i = mask_b.astype(jnp.int32)
    csum = mask_i
    for d in (1, 2, 4, 8, 16, 32, 64, 128, 256, 512, 1024):
        csum = csum + jnp.where(riota >= d, jnp.roll(csum, d, axis=0), 0)
    rank = csum - mask_i

    vrows = []
    irows = []
    for slot in range(16):
        eqb = jnp.logical_and(mask_b, rank == slot)
        vrows.append(jnp.sum(jnp.where(eqb, bits2, 0), axis=0))
        irows.append(jnp.sum(jnp.where(eqb, gidx, 0), axis=0))
    v16 = jnp.stack(vrows, axis=0)
    i16 = jnp.stack(irows, axis=0)
    si = jax.lax.broadcasted_iota(jnp.int32, (16, 128), 0)
    li = jax.lax.broadcasted_iota(jnp.int32, (16, 128), 1)
    empty = v16 == 0
    nv = jnp.where(empty, jnp.int32(1), -v16)
    ii = jnp.where(empty, (1 << 25) + si * 128 + li, i16)

    def partner(a, d):
        if d < 128:
            lo_side = (li & d) == 0
            return jnp.where(lo_side, jnp.roll(a, -d, axis=1),
                             jnp.roll(a, d, axis=1))
        ds = d // 128
        lo_side = (si & ds) == 0
        return jnp.where(lo_side, jnp.roll(a, -ds, axis=0),
                         jnp.roll(a, ds, axis=0))

    f_iota = si * 128 + li
    for k in range(1, 12):
        for j in reversed(range(k)):
            d = 1 << j
            nv_p = partner(nv, d)
            ii_p = partner(ii, d)
            is_lower = (f_iota & d) == 0
            want_min = (jax.lax.shift_right_logical(f_iota, k) & 1) == 0
            self_le = jnp.logical_or(
                nv < nv_p, jnp.logical_and(nv == nv_p, ii <= ii_p))
            keep_self = (is_lower == want_min) == self_le
            nv = jnp.where(keep_self, nv, nv_p)
            ii = jnp.where(keep_self, ii, ii_p)

    idx_sorted = ii[0:2, :]
    idx_out[...] = idx_sorted

    ys = jax.lax.div(idx_sorted, FLS_W)
    xs = jax.lax.rem(idx_sorted, FLS_W)
    cy_out[...] = jnp.clip(jax.lax.div(ys, DOWN) - PS // 2, 0, FH - PS)
    cx_out[...] = jnp.clip(jax.lax.div(xs, DOWN) - PS // 2, 0, FW - PS)
    r = (ys.astype(jnp.float32) / FLS_H) * (R_MAX - R_MIN) + R_MIN
    theta = (xs.astype(jnp.float32) / FLS_W - 0.5) * (
        FOV_H * jnp.pi / 180.0)
    phi = jnp.zeros((2, 128), jnp.float32)
    new_state = jnp.concatenate(
        [r[None], theta[None], phi[None]], axis=0)
    row3 = jax.lax.broadcasted_iota(jnp.int32, (BUFF, 3, 2, 128), 0)
    pstate_out[...] = jnp.where(row3 == local, new_state[None],
                                pstate_ref[...])

    row2 = jax.lax.broadcasted_iota(jnp.int32, (BUFF, P), 0)
    sframe_out[...] = jnp.where(row2 == local, fn, sframe_ref[...])
    lane16 = jax.lax.broadcasted_iota(jnp.int32, (1, BUFF), 1)
    time_out[...] = jnp.where(lane16 == local, ts, time_ref[...])

    k1 = jax.lax.rem(local - 1 + BUFF, BUFF)
    k2 = jax.lax.rem(local - 2 + BUFF, BUFF)
    tvec = time_ref[...]
    l16 = lane16
    t1 = jnp.sum(jnp.where(l16 == k1, tvec, 0.0))
    t2 = jnp.sum(jnp.where(l16 == k2, tvec, 0.0))
    x1 = poses_ref[pl.ds(k1, 1), :]
    x2 = poses_ref[pl.ds(k2, 1), :]
    dt_ratio = (ts - t1) / (t1 - t2)
    new_pose = x1[:, 0:3] + (x1[:, 0:3] - x2[:, 0:3]) * dt_ratio
    q1 = x1[:, 3:7]
    q2 = x2[:, 3:7]
    dot12 = jnp.sum(q1 * q2)
    q1 = jnp.where(dot12 < 0, -q1, q1)
    x1q, y1q, z1q, w1q = q1[:, 0:1], q1[:, 1:2], q1[:, 2:3], q1[:, 3:4]
    x2q, y2q, z2q, w2q = -q2[:, 0:1], -q2[:, 1:2], -q2[:, 2:3], q2[:, 3:4]
    dw = w1q * w2q - x1q * x2q - y1q * y2q - z1q * z2q
    dx = w1q * x2q + x1q * w2q + y1q * z2q - z1q * y2q
    dy = w1q * y2q - x1q * z2q + y1q * w2q + z1q * x2q
    dz = w1q * z2q + x1q * y2q - y1q * x2q + z1q * w2q
    s_ = jnp.sqrt(jnp.clip(1.0 - dw * dw, 0.0, None))
    small = s_ < 1e-3
    denom = jnp.maximum(s_, 1e-12)
    ax = jnp.where(small, 1.0, dx / denom)
    ay = jnp.where(small, 0.0, dy / denom)
    az = jnp.where(small, 0.0, dz / denom)
    dwc = jnp.clip(dw, -1.0, 1.0)
    adw = jnp.abs(dwc)
    acos_pos = jnp.sqrt(jnp.maximum(1.0 - adw, 0.0)) * (
        1.5707288 + adw * (-0.2121144 + adw * (0.0742610 - adw * 0.0187293)))
    acos_dw = jnp.where(dwc < 0, jnp.float32(jnp.pi) - acos_pos, acos_pos)
    rot_angle = 2.0 * acos_dw
    rot_a = rot_angle * dt_ratio
    sh = jnp.sin(rot_a / 2.0)
    ch = jnp.cos(rot_a / 2.0)
    qsx, qsy, qsz, qsw = ax * sh, ay * sh, az * sh, ch
    q0w = qsw * w1q - qsx * x1q - qsy * y1q - qsz * z1q
    q0x = qsw * x1q + qsx * w1q + qsy * z1q - qsz * y1q
    q0y = qsw * y1q - qsx * z1q + qsy * w1q + qsz * x1q
    q0z = qsw * z1q + qsx * y1q - qsy * x1q + qsz * w1q
    qn = jnp.sqrt(q0x * q0x + q0y * q0y + q0z * q0z + q0w * q0w)
    x0 = jnp.concatenate(
        [new_pose, q0x / qn, q0y / qn, q0z / qn, q0w / qn], axis=1)
    row7 = jax.lax.broadcasted_iota(jnp.int32, (BUFF, 7), 0)
    poses_out[...] = jnp.where(row7 == local, x0, poses_ref[...])

    lane = jax.lax.broadcasted_iota(jnp.int32, (1, SLOT), 1)
    first = lane < (T * P)
    i_new = jnp.where(first, fn * P + jax.lax.rem(lane, P),
                      (fn - T) * P + (lane - T * P))
    j_new = jnp.where(first, fn - 1 - jax.lax.div(lane, P), fn)
    rows = jax.lax.broadcasted_iota(jnp.int32, (BUFF, SLOT), 0)
    at_local = rows == local
    i_out[...] = jnp.where(at_local, i_new, i_ref[...])
    j_out[...] = jnp.where(at_local, j_new, j_ref[...])
    w_out[...] = jnp.where(at_local, 0.0, w_ref[...])
    v_out[...] = jnp.where(at_local, jnp.int8(1), v_ref[...])

    cp_f.wait()
    cp_i.wait()


def _patch_slot_body(np_ref, fn_ref, p_hbm_in, p_hbm_out, sem):
    local = jax.lax.rem(fn_ref[0], BUFF)
    for c in range(C):
        pltpu.make_async_copy(
            np_ref.at[c], p_hbm_out.at[local, :, c], sem).start()
    for c in range(C):
        pltpu.make_async_copy(
            np_ref.at[c], p_hbm_out.at[local, :, c], sem).wait()


NROWS8 = C * FH * FW // 8
PROWS = BUFF * P * C * PS * PS // 16
NW = 32
PPW = P // NW
FPW = PPW * C * PS * 2
OPW = PPW * C * PS * PS // 16


def _sc_patch_body(fmap2, cy_hbm, cx_hbm, loc_hbm, pold, pout,
                   cyv, cxv, locv, fmloc, outv, csem, ssem):
    wid = jax.lax.axis_index("s") * 2 + jax.lax.axis_index("c")
    pltpu.sync_copy(cy_hbm, cyv)
    pltpu.sync_copy(cx_hbm, cxv)
    pltpu.sync_copy(loc_hbm, locv)
    loc = jnp.minimum(jnp.max(locv[...]), BUFF - 1)
    iota16 = jax.lax.iota(jnp.int32, 16)

    pltpu.make_async_copy(
        pold.at[0, :, pl.ds(wid * 2, 2), :], outv, csem).start()
    for k in range(BUFF):
        pltpu.make_async_copy(
            pold.at[k, :, pl.ds(wid * 2, 2), :], outv, csem).wait()
        pltpu.make_async_copy(
            outv, pout.at[k, :, pl.ds(wid * 2, 2), :], csem).start()
        if k + 1 < BUFF:
            pltpu.make_async_copy(
                outv, pout.at[k, :, pl.ds(wid * 2, 2), :], csem).wait()
            pltpu.make_async_copy(
                pold.at[k + 1, :, pl.ds(wid * 2, 2), :], outv, csem).start()
    pltpu.make_async_copy(
        outv, pout.at[BUFF - 1, :, pl.ds(wid * 2, 2), :], csem).wait()

    pltpu.sync_copy(fmap2.at[pl.ds(wid * 2, 2), :], fmloc)

    zero16 = jnp.zeros((16,), jnp.int32)

    def pack(t, _):
        p = jax.lax.shift_right_logical(t, 3)
        c_loc = jax.lax.shift_right_logical(t, 2) & 1
        m = t & 3
        p16 = zero16 + p
        cy16 = plsc.load_gather(cyv, [p16])
        cx16 = plsc.load_gather(cxv, [p16])
        dy = 2 * m + jax.lax.shift_right_logical(iota16, 3)
        pos = (cy16 + dy) * FW + cx16 + (iota16 & 7)
        c16 = zero16 + c_loc
        outv[p, c_loc, pl.ds(m * 16, 16)] = plsc.load_gather(
            fmloc, [c16, pos])
        return 0

    jax.lax.fori_loop(0, P * 2 * 4, pack, 0)

    cp = pltpu.make_async_copy(
        outv, pout.at[loc, :, pl.ds(wid * 2, 2), :], ssem)
    cp.start()
    cp.wait()


def _sc_patches(fmap2, cy256, cx256, loc16, pold):
    mesh = plsc.VectorSubcoreMesh(core_axis_name="c", subcore_axis_name="s")
    fn = pl.kernel(
        _sc_patch_body,
        out_type=jax.ShapeDtypeStruct((BUFF, P, C, PS * PS), jnp.float32),
        mesh=mesh,
        compiler_params=pltpu.CompilerParams(needs_layout_passes=False),
        scratch_types=[
            pltpu.VMEM((P,), jnp.int32),
            pltpu.VMEM((P,), jnp.int32),
            pltpu.VMEM((16,), jnp.int32),
            pltpu.VMEM((2, FH * FW), jnp.float32),
            pltpu.VMEM((P, 2, PS * PS), jnp.float32),
            pltpu.SemaphoreType.DMA,
            pltpu.SemaphoreType.DMA,
        ],
    )
    return fn(fmap2, cy256, cx256, loc16, pold)


def kernel(frame, time_stamp, frame_n, W_f, W_i, time_buf, poses_buf,
           fmap_buf, imap_buf, patches_buf, patch_state, source_frame,
           i_buf, j_buf, w_buf, v_buf):
    fn1 = jnp.asarray(frame_n, jnp.int32).reshape(1)
    x = frame[0, 0]
    xbT = x.reshape(FH, DOWN, FW, DOWN).transpose(1, 3, 0, 2).reshape(
        DOWN * DOWN, FH * FW)
    sc2 = x.reshape(2048, 128)
    wcat = jnp.concatenate([W_f.T, W_i.T], axis=0)

    fmap_hbm = fmap_buf.reshape(BUFF, C, FH * FW)
    imap_hbm = imap_buf.reshape(BUFF, C, FH * FW)
    pstate_t = patch_state.transpose(0, 2, 1).reshape(BUFF, 3, 2, 128)
    time2 = time_buf.reshape(1, BUFF)
    i2 = i_buf.reshape(BUFF, SLOT)
    j2 = j_buf.reshape(BUFF, SLOT)
    w2 = w_buf.reshape(BUFF, SLOT)
    v2 = v_buf.reshape(BUFF, SLOT).astype(jnp.int8)

    vm = pltpu.MemorySpace.VMEM
    hb = pltpu.MemorySpace.HBM
    sm = pltpu.MemorySpace.SMEM
    outs = pl.pallas_call(
        _body,
        in_specs=[
            pl.BlockSpec(memory_space=vm),
            pl.BlockSpec(memory_space=vm),
            pl.BlockSpec(memory_space=vm),
            pl.BlockSpec(memory_space=sm),
            pl.BlockSpec(memory_space=sm),
            pl.BlockSpec(memory_space=vm),
            pl.BlockSpec(memory_space=vm),
            pl.BlockSpec(memory_space=vm),
            pl.BlockSpec(memory_space=vm),
            pl.BlockSpec(memory_space=vm),
            pl.BlockSpec(memory_space=vm),
            pl.BlockSpec(memory_space=vm),
            pl.BlockSpec(memory_space=vm),
            pl.BlockSpec(memory_space=hb),
            pl.BlockSpec(memory_space=hb),
        ],
        out_specs=[
            pl.BlockSpec(memory_space=hb),
            pl.BlockSpec(memory_space=hb),
            pl.BlockSpec(memory_space=vm),
            pl.BlockSpec(memory_space=vm),
            pl.BlockSpec(memory_space=vm),
            pl.BlockSpec(memory_space=vm),
            pl.BlockSpec(memory_space=vm),
            pl.BlockSpec(memory_space=vm),
            pl.BlockSpec(memory_space=vm),
            pl.BlockSpec(memory_space=vm),
            pl.BlockSpec(memory_space=vm),
            pl.BlockSpec(memory_space=vm),
            pl.BlockSpec(memory_space=vm),
            pl.BlockSpec(memory_space=vm),
        ],
        out_shape=[
            jax.ShapeDtypeStruct((BUFF, C, FH * FW), jnp.float32),
            jax.ShapeDtypeStruct((BUFF, C, FH * FW), jnp.float32),
            jax.ShapeDtypeStruct((C, FH * FW), jnp.float32),
            jax.ShapeDtypeStruct((1, BUFF), jnp.float32),
            jax.ShapeDtypeStruct((BUFF, 7), jnp.float32),
            jax.ShapeDtypeStruct((BUFF, 3, 2, 128), jnp.float32),
            jax.ShapeDtypeStruct((BUFF, P), jnp.int32),
            jax.ShapeDtypeStruct((BUFF, SLOT), jnp.int32),
            jax.ShapeDtypeStruct((BUFF, SLOT), jnp.int32),
            jax.ShapeDtypeStruct((BUFF, SLOT), jnp.float32),
            jax.ShapeDtypeStruct((BUFF, SLOT), jnp.int8),
            jax.ShapeDtypeStruct((2, 128), jnp.int32),
            jax.ShapeDtypeStruct((2, 128), jnp.int32),
            jax.ShapeDtypeStruct((2, 128), jnp.int32),
        ],
        scratch_shapes=[
            pltpu.VMEM((C, FH * FW), jnp.float32),
            pltpu.SemaphoreType.DMA,
            pltpu.SemaphoreType.DMA,
        ],
        input_output_aliases={13: 0, 14: 1},
    )(xbT, sc2, wcat, fn1, time_stamp, time2, poses_buf, pstate_t,
      source_frame, i2, j2, w2, v2, fmap_hbm, imap_hbm)

    (fmap_o, imap_o, fmap_s, time_o, poses_o, pstate_o, sframe_o,
     i_o, j_o, w_o, v_o, idx_o, cy_o, cx_o) = outs

    fn_i = jnp.asarray(frame_n, jnp.int32)
    local = fn_i % BUFF
    cy256 = cy_o.reshape(P)
    cx256 = cx_o.reshape(P)
    loc16 = jnp.full((16,), local, jnp.int32)
    pold = patches_buf.reshape(BUFF, P, C, PS * PS)
    patches_o = _sc_patches(fmap_s, cy256, cx256, loc16, pold).reshape(
        BUFF, P, C, PS, PS)

    return (fmap_o.reshape(BUFF, C, FH, FW),
            imap_o.reshape(BUFF, C, FH, FW),
            patches_o,
            pstate_o.reshape(BUFF, 3, P).transpose(0, 2, 1),
            poses_o,
            time_o.reshape(BUFF),
            sframe_o,
            i_o.reshape(MAX_EDGES),
            j_o.reshape(MAX_EDGES),
            w_o.reshape(MAX_EDGES),
            (v_o != 0).reshape(MAX_EDGES))


# speedup vs baseline: 1.5448x; 1.5448x over previous
import jax
import jax.numpy as jnp
from jax.experimental import pallas as pl
from jax.experimental.pallas import tpu as pltpu
from jax.experimental.pallas import tpu_sc as plsc
import functools

R_MIN = 0.5
R_MAX = 30.0
FLS_H = 512
FLS_W = 512
FOV_H = 130.0
BUFF = 16
P = 256
PS = 8
T = 8
C = 64
DOWN = 4
FH = FLS_H // DOWN
FW = FLS_W // DOWN
SLOT = 2 * P * T
MAX_EDGES = BUFF * SLOT
NPIX = FLS_H * FLS_W
NBLK = 256
KCAND = 512


def _body(xbT_ref, sc3_ref, wcat_ref, fn_ref, ts_ref, time_ref, poses_ref,
          pstate_ref, sframe_ref, i_ref, j_ref, w_ref, v_ref,
          fmap_hbm_in, imap_hbm_in,
          fmap_hbm_out, imap_hbm_out, fmap_out, time_out, poses_out,
          pstate_out, sframe_out, i_out, j_out, w_out, v_out, idx_out,
          cy_out, cx_out,
          imap_s, sem_f, sem_i):
    fn = fn_ref[0]
    local = jax.lax.rem(fn, BUFF)
    ts = ts_ref[0]

    both = jnp.dot(wcat_ref[...], xbT_ref[...],
                   preferred_element_type=jnp.float32)
    fmap_out[...] = both[:C]
    imap_s[...] = jnp.tanh(both[C:])
    cp_f = pltpu.make_async_copy(fmap_out, fmap_hbm_out.at[local], sem_f)
    cp_i = pltpu.make_async_copy(imap_s, imap_hbm_out.at[local], sem_i)
    cp_f.start()
    cp_i.start()

    s2 = sc3_ref[...]
    bits2 = jax.lax.bitcast_convert_type(s2 * s2, jnp.int32)

    def bisect(_, carry):
        lo, hi = carry
        mid = lo + jax.lax.div(hi - lo, 2)
        cnt = jnp.sum((bits2 >= mid).astype(jnp.int32))
        big = cnt >= P
        return (jnp.where(big, mid, lo), jnp.where(big, hi, mid))

    t_bits, _ = jax.lax.fori_loop(0, 31, bisect, (jnp.int32(0),
                                                  jnp.int32(0x7F800000)))

    riota = jax.lax.broadcasted_iota(jnp.int32, (2048, 128), 0)
    liota = jax.lax.broadcasted_iota(jnp.int32, (2048, 128), 1)
    gidx = riota * 128 + liota
    mask_b = bits2 >= t_bits
    mask_i = mask_b.astype(jnp.int32)
    csum = mask_i
    for d in (1, 2, 4, 8, 16, 32, 64, 128, 256, 512, 1024):
        csum = csum + jnp.where(riota >= d, jnp.roll(csum, d, axis=0), 0)
    rank = csum - mask_i

    vrows = []
    irows = []
    for slot in range(16):
        eqb = jnp.logical_and(mask_b, rank == slot)
        vrows.append(jnp.sum(jnp.where(eqb, bits2, 0), axis=0))
        irows.append(jnp.sum(jnp.where(eqb, gidx, 0), axis=0))
    v16 = jnp.stack(vrows, axis=0)
    i16 = jnp.stack(irows, axis=0)
    si = jax.lax.broadcasted_iota(jnp.int32, (16, 128), 0)
    li = jax.lax.broadcasted_iota(jnp.int32, (16, 128), 1)
    empty = v16 == 0
    nv = jnp.where(empty, jnp.int32(1), -v16)
    ii = jnp.where(empty, (1 << 25) + si * 128 + li, i16)

    def partner(a, d):
        if d < 128:
            lo_side = (li & d) == 0
            return jnp.where(lo_side, jnp.roll(a, -d, axis=1),
                             jnp.roll(a, d, axis=1))
        ds = d // 128
        lo_side = (si & ds) == 0
        return jnp.where(lo_side, jnp.roll(a, -ds, axis=0),
                         jnp.roll(a, ds, axis=0))

    f_iota = si * 128 + li
    for k in range(1, 12):
        for j in reversed(range(k)):
            d = 1 << j
            nv_p = partner(nv, d)
            ii_p = partner(ii, d)
            is_lower = (f_iota & d) == 0
            want_min = (jax.lax.shift_right_logical(f_iota, k) & 1) == 0
            self_le = jnp.logical_or(
                nv < nv_p, jnp.logical_and(nv == nv_p, ii <= ii_p))
            keep_self = (is_lower == want_min) == self_le
            nv = jnp.where(keep_self, nv, nv_p)
            ii = jnp.where(keep_self, ii, ii_p)

    idx_sorted = ii[0:2, :]
    idx_out[...] = idx_sorted

    ys = jax.lax.div(idx_sorted, FLS_W)
    xs = jax.lax.rem(idx_sorted, FLS_W)
    cy_out[...] = jnp.clip(jax.lax.div(ys, DOWN) - PS // 2, 0, FH - PS)
    cx_out[...] = jnp.clip(jax.lax.div(xs, DOWN) - PS // 2, 0, FW - PS)
    r = (ys.astype(jnp.float32) / FLS_H) * (R_MAX - R_MIN) + R_MIN
    theta = (xs.astype(jnp.float32) / FLS_W - 0.5) * (
        FOV_H * jnp.pi / 180.0)
    phi = jnp.zeros((2, 128), jnp.float32)
    new_state = jnp.concatenate(
        [r[None], theta[None], phi[None]], axis=0)
    row3 = jax.lax.broadcasted_iota(jnp.int32, (BUFF, 3, 2, 128), 0)
    pstate_out[...] = jnp.where(row3 == local, new_state[None],
                                pstate_ref[...])

    row2 = jax.lax.broadcasted_iota(jnp.int32, (BUFF, P), 0)
    sframe_out[...] = jnp.where(row2 == local, fn, sframe_ref[...])
    lane16 = jax.lax.broadcasted_iota(jnp.int32, (1, BUFF), 1)
    time_out[...] = jnp.where(lane16 == local, ts, time_ref[...])

    k1 = jax.lax.rem(local - 1 + BUFF, BUFF)
    k2 = jax.lax.rem(local - 2 + BUFF, BUFF)
    tvec = time_ref[...]
    l16 = lane16
    t1 = jnp.sum(jnp.where(l16 == k1, tvec, 0.0))
    t2 = jnp.sum(jnp.where(l16 == k2, tvec, 0.0))
    x1 = poses_ref[pl.ds(k1, 1), :]
    x2 = poses_ref[pl.ds(k2, 1), :]
    dt_ratio = (ts - t1) / (t1 - t2)
    new_pose = x1[:, 0:3] + (x1[:, 0:3] - x2[:, 0:3]) * dt_ratio
    q1 = x1[:, 3:7]
    q2 = x2[:, 3:7]
    dot12 = jnp.sum(q1 * q2)
    q1 = jnp.where(dot12 < 0, -q1, q1)
    x1q, y1q, z1q, w1q = q1[:, 0:1], q1[:, 1:2], q1[:, 2:3], q1[:, 3:4]
    x2q, y2q, z2q, w2q = -q2[:, 0:1], -q2[:, 1:2], -q2[:, 2:3], q2[:, 3:4]
    dw = w1q * w2q - x1q * x2q - y1q * y2q - z1q * z2q
    dx = w1q * x2q + x1q * w2q + y1q * z2q - z1q * y2q
    dy = w1q * y2q - x1q * z2q + y1q * w2q + z1q * x2q
    dz = w1q * z2q + x1q * y2q - y1q * x2q + z1q * w2q
    s_ = jnp.sqrt(jnp.clip(1.0 - dw * dw, 0.0, None))
    small = s_ < 1e-3
    denom = jnp.maximum(s_, 1e-12)
    ax = jnp.where(small, 1.0, dx / denom)
    ay = jnp.where(small, 0.0, dy / denom)
    az = jnp.where(small, 0.0, dz / denom)
    dwc = jnp.clip(dw, -1.0, 1.0)
    adw = jnp.abs(dwc)
    acos_pos = jnp.sqrt(jnp.maximum(1.0 - adw, 0.0)) * (
        1.5707288 + adw * (-0.2121144 + adw * (0.0742610 - adw * 0.0187293)))
    acos_dw = jnp.where(dwc < 0, jnp.float32(jnp.pi) - acos_pos, acos_pos)
    rot_angle = 2.0 * acos_dw
    rot_a = rot_angle * dt_ratio
    sh = jnp.sin(rot_a / 2.0)
    ch = jnp.cos(rot_a / 2.0)
    qsx, qsy, qsz, qsw = ax * sh, ay * sh, az * sh, ch
    q0w = qsw * w1q - qsx * x1q - qsy * y1q - qsz * z1q
    q0x = qsw * x1q + qsx * w1q + qsy * z1q - qsz * y1q
    q0y = qsw * y1q - qsx * z1q + qsy * w1q + qsz * x1q
    q0z = qsw * z1q + qsx * y1q - qsy * x1q + qsz * w1q
    qn = jnp.sqrt(q0x * q0x + q0y * q0y + q0z * q0z + q0w * q0w)
    x0 = jnp.concatenate(
        [new_pose, q0x / qn, q0y / qn, q0z / qn, q0w / qn], axis=1)
    row7 = jax.lax.broadcasted_iota(jnp.int32, (BUFF, 7), 0)
    poses_out[...] = jnp.where(row7 == local, x0, poses_ref[...])

    lane = jax.lax.broadcasted_iota(jnp.int32, (1, SLOT), 1)
    first = lane < (T * P)
    i_new = jnp.where(first, fn * P + jax.lax.rem(lane, P),
                      (fn - T) * P + (lane - T * P))
    j_new = jnp.where(first, fn - 1 - jax.lax.div(lane, P), fn)
    rows = jax.lax.broadcasted_iota(jnp.int32, (BUFF, SLOT), 0)
    at_local = rows == local
    i_out[...] = jnp.where(at_local, i_new, i_ref[...])
    j_out[...] = jnp.where(at_local, j_new, j_ref[...])
    w_out[...] = jnp.where(at_local, 0.0, w_ref[...])
    v_out[...] = jnp.where(at_local, jnp.int8(1), v_ref[...])

    cp_f.wait()
    cp_i.wait()


def _patch_slot_body(np_ref, fn_ref, p_hbm_in, p_hbm_out, sem):
    local = jax.lax.rem(fn_ref[0], BUFF)
    for c in range(C):
        pltpu.make_async_copy(
            np_ref.at[c], p_hbm_out.at[local, :, c], sem).start()
    for c in range(C):
        pltpu.make_async_copy(
            np_ref.at[c], p_hbm_out.at[local, :, c], sem).wait()


NROWS8 = C * FH * FW // 8
PROWS = BUFF * P * C * PS * PS // 16
NW = 32
PPW = P // NW
FPW = PPW * C * PS * 2
OPW = PPW * C * PS * PS // 16


def _sc_patch_body(fmap2, cy_hbm, cx_hbm, pout,
                   cyv, cxv, fmloc, outv, ssem):
    wid = jax.lax.axis_index("s") * 2 + jax.lax.axis_index("c")
    pltpu.sync_copy(cy_hbm, cyv)
    pltpu.sync_copy(cx_hbm, cxv)
    iota16 = jax.lax.iota(jnp.int32, 16)

    pltpu.sync_copy(fmap2.at[pl.ds(wid * 2, 2), :], fmloc)

    zero16 = jnp.zeros((16,), jnp.int32)

    def pack(t, _):
        p = jax.lax.shift_right_logical(t, 3)
        c_loc = jax.lax.shift_right_logical(t, 2) & 1
        m = t & 3
        p16 = zero16 + p
        cy16 = plsc.load_gather(cyv, [p16])
        cx16 = plsc.load_gather(cxv, [p16])
        dy = 2 * m + jax.lax.shift_right_logical(iota16, 3)
        pos = (cy16 + dy) * FW + cx16 + (iota16 & 7)
        c16 = zero16 + c_loc
        outv[p, c_loc, pl.ds(m * 16, 16)] = plsc.load_gather(
            fmloc, [c16, pos])
        return 0

    jax.lax.fori_loop(0, P * 2 * 4, pack, 0)

    cp = pltpu.make_async_copy(
        outv, pout.at[:, pl.ds(wid * 2, 2), :], ssem)
    cp.start()
    cp.wait()


def _sc_patches(fmap2, cy256, cx256):
    mesh = plsc.VectorSubcoreMesh(core_axis_name="c", subcore_axis_name="s")
    fn = pl.kernel(
        _sc_patch_body,
        out_type=jax.ShapeDtypeStruct((P, C, PS * PS), jnp.float32),
        mesh=mesh,
        compiler_params=pltpu.CompilerParams(needs_layout_passes=False),
        scratch_types=[
            pltpu.VMEM((P,), jnp.int32),
            pltpu.VMEM((P,), jnp.int32),
            pltpu.VMEM((2, FH * FW), jnp.float32),
            pltpu.VMEM((P, 2, PS * PS), jnp.float32),
            pltpu.SemaphoreType.DMA,
        ],
    )
    return fn(fmap2, cy256, cx256)


def kernel(frame, time_stamp, frame_n, W_f, W_i, time_buf, poses_buf,
           fmap_buf, imap_buf, patches_buf, patch_state, source_frame,
           i_buf, j_buf, w_buf, v_buf):
    fn1 = jnp.asarray(frame_n, jnp.int32).reshape(1)
    x = frame[0, 0]
    xbT = x.reshape(FH, DOWN, FW, DOWN).transpose(1, 3, 0, 2).reshape(
        DOWN * DOWN, FH * FW)
    sc2 = x.reshape(2048, 128)
    wcat = jnp.concatenate([W_f.T, W_i.T], axis=0)

    fmap_hbm = fmap_buf.reshape(BUFF, C, FH * FW)
    imap_hbm = imap_buf.reshape(BUFF, C, FH * FW)
    pstate_t = patch_state.transpose(0, 2, 1).reshape(BUFF, 3, 2, 128)
    time2 = time_buf.reshape(1, BUFF)
    i2 = i_buf.reshape(BUFF, SLOT)
    j2 = j_buf.reshape(BUFF, SLOT)
    w2 = w_buf.reshape(BUFF, SLOT)
    v2 = v_buf.reshape(BUFF, SLOT).astype(jnp.int8)

    vm = pltpu.MemorySpace.VMEM
    hb = pltpu.MemorySpace.HBM
    sm = pltpu.MemorySpace.SMEM
    outs = pl.pallas_call(
        _body,
        in_specs=[
            pl.BlockSpec(memory_space=vm),
            pl.BlockSpec(memory_space=vm),
            pl.BlockSpec(memory_space=vm),
            pl.BlockSpec(memory_space=sm),
            pl.BlockSpec(memory_space=sm),
            pl.BlockSpec(memory_space=vm),
            pl.BlockSpec(memory_space=vm),
            pl.BlockSpec(memory_space=vm),
            pl.BlockSpec(memory_space=vm),
            pl.BlockSpec(memory_space=vm),
            pl.BlockSpec(memory_space=vm),
            pl.BlockSpec(memory_space=vm),
            pl.BlockSpec(memory_space=vm),
            pl.BlockSpec(memory_space=hb),
            pl.BlockSpec(memory_space=hb),
        ],
        out_specs=[
            pl.BlockSpec(memory_space=hb),
            pl.BlockSpec(memory_space=hb),
            pl.BlockSpec(memory_space=vm),
            pl.BlockSpec(memory_space=vm),
            pl.BlockSpec(memory_space=vm),
            pl.BlockSpec(memory_space=vm),
            pl.BlockSpec(memory_space=vm),
            pl.BlockSpec(memory_space=vm),
            pl.BlockSpec(memory_space=vm),
            pl.BlockSpec(memory_space=vm),
            pl.BlockSpec(memory_space=vm),
            pl.BlockSpec(memory_space=vm),
            pl.BlockSpec(memory_space=vm),
            pl.BlockSpec(memory_space=vm),
        ],
        out_shape=[
            jax.ShapeDtypeStruct((BUFF, C, FH * FW), jnp.float32),
            jax.ShapeDtypeStruct((BUFF, C, FH * FW), jnp.float32),
            jax.ShapeDtypeStruct((C, FH * FW), jnp.float32),
            jax.ShapeDtypeStruct((1, BUFF), jnp.float32),
            jax.ShapeDtypeStruct((BUFF, 7), jnp.float32),
            jax.ShapeDtypeStruct((BUFF, 3, 2, 128), jnp.float32),
            jax.ShapeDtypeStruct((BUFF, P), jnp.int32),
            jax.ShapeDtypeStruct((BUFF, SLOT), jnp.int32),
            jax.ShapeDtypeStruct((BUFF, SLOT), jnp.int32),
            jax.ShapeDtypeStruct((BUFF, SLOT), jnp.float32),
            jax.ShapeDtypeStruct((BUFF, SLOT), jnp.int8),
            jax.ShapeDtypeStruct((2, 128), jnp.int32),
            jax.ShapeDtypeStruct((2, 128), jnp.int32),
            jax.ShapeDtypeStruct((2, 128), jnp.int32),
        ],
        scratch_shapes=[
            pltpu.VMEM((C, FH * FW), jnp.float32),
            pltpu.SemaphoreType.DMA,
            pltpu.SemaphoreType.DMA,
        ],
        input_output_aliases={13: 0, 14: 1},
    )(xbT, sc2, wcat, fn1, time_stamp, time2, poses_buf, pstate_t,
      source_frame, i2, j2, w2, v2, fmap_hbm, imap_hbm)

    (fmap_o, imap_o, fmap_s, time_o, poses_o, pstate_o, sframe_o,
     i_o, j_o, w_o, v_o, idx_o, cy_o, cx_o) = outs

    fn_i = jnp.asarray(frame_n, jnp.int32)
    local = fn_i % BUFF
    cy256 = cy_o.reshape(P)
    cx256 = cx_o.reshape(P)
    new_patches = _sc_patches(fmap_s, cy256, cx256).reshape(P, C, PS, PS)
    patches_o = patches_buf.at[local].set(new_patches)

    return (fmap_o.reshape(BUFF, C, FH, FW),
            imap_o.reshape(BUFF, C, FH, FW),
            patches_o,
            pstate_o.reshape(BUFF, 3, P).transpose(0, 2, 1),
            poses_o,
            time_o.reshape(BUFF),
            sframe_o,
            i_o.reshape(MAX_EDGES),
            j_o.reshape(MAX_EDGES),
            w_o.reshape(MAX_EDGES),
            (v_o != 0).reshape(MAX_EDGES))


# speedup vs baseline: 1.5553x; 1.0068x over previous
import jax
import jax.numpy as jnp
from jax.experimental import pallas as pl
from jax.experimental.pallas import tpu as pltpu
from jax.experimental.pallas import tpu_sc as plsc
import functools

R_MIN = 0.5
R_MAX = 30.0
FLS_H = 512
FLS_W = 512
FOV_H = 130.0
BUFF = 16
P = 256
PS = 8
T = 8
C = 64
DOWN = 4
FH = FLS_H // DOWN
FW = FLS_W // DOWN
SLOT = 2 * P * T
MAX_EDGES = BUFF * SLOT
NPIX = FLS_H * FLS_W
NBLK = 256
KCAND = 512


def _body(xbT_ref, sc3_ref, wcat_ref, fn_ref, ts_ref, time_ref, poses_ref,
          pstate_ref, sframe_ref, i_ref, j_ref, w_ref, v_ref,
          fmap_hbm_in, imap_hbm_in,
          fmap_hbm_out, imap_hbm_out, fmap_out, time_out, poses_out,
          pstate_out, sframe_out, i_out, j_out, w_out, v_out, idx_out,
          cy_out, cx_out,
          imap_s, sem_f, sem_i):
    fn = fn_ref[0]
    local = jax.lax.rem(fn, BUFF)
    ts = ts_ref[0]

    both = jnp.dot(wcat_ref[...], xbT_ref[...],
                   preferred_element_type=jnp.float32)
    fmap_out[...] = both[:C]
    imap_s[...] = jnp.tanh(both[C:])
    cp_f = pltpu.make_async_copy(fmap_out, fmap_hbm_out.at[local], sem_f)
    cp_i = pltpu.make_async_copy(imap_s, imap_hbm_out.at[local], sem_i)
    cp_f.start()
    cp_i.start()

    s2 = sc3_ref[...]
    bits2 = jax.lax.bitcast_convert_type(s2 * s2, jnp.int32)

    def bisect(_, carry):
        lo, hi = carry
        mid = lo + jax.lax.div(hi - lo, 2)
        cnt = jnp.sum((bits2 >= mid).astype(jnp.int32))
        big = cnt >= P
        return (jnp.where(big, mid, lo), jnp.where(big, hi, mid))

    t_bits, _ = jax.lax.fori_loop(0, 31, bisect, (jnp.int32(0),
                                                  jnp.int32(0x7F800000)))

    riota = jax.lax.broadcasted_iota(jnp.int32, (2048, 128), 0)
    liota = jax.lax.broadcasted_iota(jnp.int32, (2048, 128), 1)
    gidx = riota * 128 + liota
    mask_b = bits2 >= t_bits
    mask_i = mask_b.astype(jnp.int32)
    csum = mask_i
    for d in (1, 2, 4, 8, 16, 32, 64, 128, 256, 512, 1024):
        csum = csum + jnp.where(riota >= d, jnp.roll(csum, d, axis=0), 0)
    rank = csum - mask_i

    vrows = []
    irows = []
    for slot in range(16):
        eqb = jnp.logical_and(mask_b, rank == slot)
        vrows.append(jnp.sum(jnp.where(eqb, bits2, 0), axis=0))
        irows.append(jnp.sum(jnp.where(eqb, gidx, 0), axis=0))
    v16 = jnp.stack(vrows, axis=0)
    i16 = jnp.stack(irows, axis=0)
    si = jax.lax.broadcasted_iota(jnp.int32, (16, 128), 0)
    li = jax.lax.broadcasted_iota(jnp.int32, (16, 128), 1)
    empty = v16 == 0
    nv = jnp.where(empty, jnp.int32(1), -v16)
    ii = jnp.where(empty, (1 << 25) + si * 128 + li, i16)

    def partner(a, d):
        if d < 128:
            lo_side = (li & d) == 0
            return jnp.where(lo_side, jnp.roll(a, -d, axis=1),
                             jnp.roll(a, d, axis=1))
        ds = d // 128
        lo_side = (si & ds) == 0
        return jnp.where(lo_side, jnp.roll(a, -ds, axis=0),
                         jnp.roll(a, ds, axis=0))

    f_iota = si * 128 + li
    for k in range(1, 12):
        for j in reversed(range(k)):
            d = 1 << j
            nv_p = partner(nv, d)
            ii_p = partner(ii, d)
            is_lower = (f_iota & d) == 0
            want_min = (jax.lax.shift_right_logical(f_iota, k) & 1) == 0
            self_le = jnp.logical_or(
                nv < nv_p, jnp.logical_and(nv == nv_p, ii <= ii_p))
            keep_self = (is_lower == want_min) == self_le
            nv = jnp.where(keep_self, nv, nv_p)
            ii = jnp.where(keep_self, ii, ii_p)

    idx_sorted = ii[0:2, :]
    idx_out[...] = idx_sorted

    ys = jax.lax.div(idx_sorted, FLS_W)
    xs = jax.lax.rem(idx_sorted, FLS_W)
    cy_out[...] = jnp.clip(jax.lax.div(ys, DOWN) - PS // 2, 0, FH - PS)
    cx_out[...] = jnp.clip(jax.lax.div(xs, DOWN) - PS // 2, 0, FW - PS)
    r = (ys.astype(jnp.float32) / FLS_H) * (R_MAX - R_MIN) + R_MIN
    theta = (xs.astype(jnp.float32) / FLS_W - 0.5) * (
        FOV_H * jnp.pi / 180.0)
    phi = jnp.zeros((2, 128), jnp.float32)
    new_state = jnp.concatenate(
        [r[None], theta[None], phi[None]], axis=0)
    row3 = jax.lax.broadcasted_iota(jnp.int32, (BUFF, 3, 2, 128), 0)
    pstate_out[...] = jnp.where(row3 == local, new_state[None],
                                pstate_ref[...])

    row2 = jax.lax.broadcasted_iota(jnp.int32, (BUFF, P), 0)
    sframe_out[...] = jnp.where(row2 == local, fn, sframe_ref[...])
    lane16 = jax.lax.broadcasted_iota(jnp.int32, (1, BUFF), 1)
    time_out[...] = jnp.where(lane16 == local, ts, time_ref[...])

    k1 = jax.lax.rem(local - 1 + BUFF, BUFF)
    k2 = jax.lax.rem(local - 2 + BUFF, BUFF)
    tvec = time_ref[...]
    l16 = lane16
    t1 = jnp.sum(jnp.where(l16 == k1, tvec, 0.0))
    t2 = jnp.sum(jnp.where(l16 == k2, tvec, 0.0))
    x1 = poses_ref[pl.ds(k1, 1), :]
    x2 = poses_ref[pl.ds(k2, 1), :]
    dt_ratio = (ts - t1) / (t1 - t2)
    new_pose = x1[:, 0:3] + (x1[:, 0:3] - x2[:, 0:3]) * dt_ratio
    q1 = x1[:, 3:7]
    q2 = x2[:, 3:7]
    dot12 = jnp.sum(q1 * q2)
    q1 = jnp.where(dot12 < 0, -q1, q1)
    x1q, y1q, z1q, w1q = q1[:, 0:1], q1[:, 1:2], q1[:, 2:3], q1[:, 3:4]
    x2q, y2q, z2q, w2q = -q2[:, 0:1], -q2[:, 1:2], -q2[:, 2:3], q2[:, 3:4]
    dw = w1q * w2q - x1q * x2q - y1q * y2q - z1q * z2q
    dx = w1q * x2q + x1q * w2q + y1q * z2q - z1q * y2q
    dy = w1q * y2q - x1q * z2q + y1q * w2q + z1q * x2q
    dz = w1q * z2q + x1q * y2q - y1q * x2q + z1q * w2q
    s_ = jnp.sqrt(jnp.clip(1.0 - dw * dw, 0.0, None))
    small = s_ < 1e-3
    denom = jnp.maximum(s_, 1e-12)
    ax = jnp.where(small, 1.0, dx / denom)
    ay = jnp.where(small, 0.0, dy / denom)
    az = jnp.where(small, 0.0, dz / denom)
    dwc = jnp.clip(dw, -1.0, 1.0)
    adw = jnp.abs(dwc)
    acos_pos = jnp.sqrt(jnp.maximum(1.0 - adw, 0.0)) * (
        1.5707288 + adw * (-0.2121144 + adw * (0.0742610 - adw * 0.0187293)))
    acos_dw = jnp.where(dwc < 0, jnp.float32(jnp.pi) - acos_pos, acos_pos)
    rot_angle = 2.0 * acos_dw
    rot_a = rot_angle * dt_ratio
    sh = jnp.sin(rot_a / 2.0)
    ch = jnp.cos(rot_a / 2.0)
    qsx, qsy, qsz, qsw = ax * sh, ay * sh, az * sh, ch
    q0w = qsw * w1q - qsx * x1q - qsy * y1q - qsz * z1q
    q0x = qsw * x1q + qsx * w1q + qsy * z1q - qsz * y1q
    q0y = qsw * y1q - qsx * z1q + qsy * w1q + qsz * x1q
    q0z = qsw * z1q + qsx * y1q - qsy * x1q + qsz * w1q
    qn = jnp.sqrt(q0x * q0x + q0y * q0y + q0z * q0z + q0w * q0w)
    x0 = jnp.concatenate(
        [new_pose, q0x / qn, q0y / qn, q0z / qn, q0w / qn], axis=1)
    row7 = jax.lax.broadcasted_iota(jnp.int32, (BUFF, 7), 0)
    poses_out[...] = jnp.where(row7 == local, x0, poses_ref[...])

    lane = jax.lax.broadcasted_iota(jnp.int32, (1, SLOT), 1)
    first = lane < (T * P)
    i_new = jnp.where(first, fn * P + jax.lax.rem(lane, P),
                      (fn - T) * P + (lane - T * P))
    j_new = jnp.where(first, fn - 1 - jax.lax.div(lane, P), fn)
    rows = jax.lax.broadcasted_iota(jnp.int32, (BUFF, SLOT), 0)
    at_local = rows == local
    i_out[...] = jnp.where(at_local, i_new, i_ref[...])
    j_out[...] = jnp.where(at_local, j_new, j_ref[...])
    w_out[...] = jnp.where(at_local, 0.0, w_ref[...])
    v_out[...] = jnp.where(at_local, jnp.int8(1), v_ref[...])

    cp_f.wait()
    cp_i.wait()


def _patch_slot_body(np_ref, fn_ref, p_hbm_in, p_hbm_out, sem):
    local = jax.lax.rem(fn_ref[0], BUFF)
    for c in range(C):
        pltpu.make_async_copy(
            np_ref.at[c], p_hbm_out.at[local, :, c], sem).start()
    for c in range(C):
        pltpu.make_async_copy(
            np_ref.at[c], p_hbm_out.at[local, :, c], sem).wait()


NROWS8 = C * FH * FW // 8
PROWS = BUFF * P * C * PS * PS // 16
NW = 32
PPW = P // NW
FPW = PPW * C * PS * 2
OPW = PPW * C * PS * PS // 16


def _sc_patch_body(fmap2, cy_hbm, cx_hbm, pout,
                   cyv, cxv, fmloc, outv, ssem):
    wid = jax.lax.axis_index("s") * 2 + jax.lax.axis_index("c")
    pltpu.sync_copy(cy_hbm, cyv)
    pltpu.sync_copy(cx_hbm, cxv)
    iota16 = jax.lax.iota(jnp.int32, 16)

    pltpu.sync_copy(fmap2.at[pl.ds(wid * 2, 2), :], fmloc)

    zero16 = jnp.zeros((16,), jnp.int32)

    for p0 in range(0, P, 16):
        cy16 = cyv[pl.ds(p0, 16)]
        cx16 = cxv[pl.ds(p0, 16)]
        cyv[pl.ds(p0, 16)] = cy16 * FW + cx16

    lane_off = (jax.lax.shift_right_logical(iota16, 3) * FW
                + (iota16 & 7))

    def pack(t, _):
        p = jax.lax.shift_right_logical(t, 3)
        c_loc = jax.lax.shift_right_logical(t, 2) & 1
        m = t & 3
        p16 = zero16 + p
        base16 = plsc.load_gather(cyv, [p16])
        pos = base16 + 2 * m * FW + lane_off
        c16 = zero16 + c_loc
        outv[p, c_loc, pl.ds(m * 16, 16)] = plsc.load_gather(
            fmloc, [c16, pos])
        return 0

    jax.lax.fori_loop(0, P * 2 * 4, pack, 0)

    cp = pltpu.make_async_copy(
        outv, pout.at[:, pl.ds(wid * 2, 2), :], ssem)
    cp.start()
    cp.wait()


def _sc_patches(fmap2, cy256, cx256):
    mesh = plsc.VectorSubcoreMesh(core_axis_name="c", subcore_axis_name="s")
    fn = pl.kernel(
        _sc_patch_body,
        out_type=jax.ShapeDtypeStruct((P, C, PS * PS), jnp.float32),
        mesh=mesh,
        compiler_params=pltpu.CompilerParams(needs_layout_passes=False),
        scratch_types=[
            pltpu.VMEM((P,), jnp.int32),
            pltpu.VMEM((P,), jnp.int32),
            pltpu.VMEM((2, FH * FW), jnp.float32),
            pltpu.VMEM((P, 2, PS * PS), jnp.float32),
            pltpu.SemaphoreType.DMA,
        ],
    )
    return fn(fmap2, cy256, cx256)


def kernel(frame, time_stamp, frame_n, W_f, W_i, time_buf, poses_buf,
           fmap_buf, imap_buf, patches_buf, patch_state, source_frame,
           i_buf, j_buf, w_buf, v_buf):
    fn1 = jnp.asarray(frame_n, jnp.int32).reshape(1)
    x = frame[0, 0]
    xbT = x.reshape(FH, DOWN, FW, DOWN).transpose(1, 3, 0, 2).reshape(
        DOWN * DOWN, FH * FW)
    sc2 = x.reshape(2048, 128)
    wcat = jnp.concatenate([W_f.T, W_i.T], axis=0)

    fmap_hbm = fmap_buf.reshape(BUFF, C, FH * FW)
    imap_hbm = imap_buf.reshape(BUFF, C, FH * FW)
    pstate_t = patch_state.transpose(0, 2, 1).reshape(BUFF, 3, 2, 128)
    time2 = time_buf.reshape(1, BUFF)
    i2 = i_buf.reshape(BUFF, SLOT)
    j2 = j_buf.reshape(BUFF, SLOT)
    w2 = w_buf.reshape(BUFF, SLOT)
    v2 = v_buf.reshape(BUFF, SLOT).astype(jnp.int8)

    vm = pltpu.MemorySpace.VMEM
    hb = pltpu.MemorySpace.HBM
    sm = pltpu.MemorySpace.SMEM
    outs = pl.pallas_call(
        _body,
        in_specs=[
            pl.BlockSpec(memory_space=vm),
            pl.BlockSpec(memory_space=vm),
            pl.BlockSpec(memory_space=vm),
            pl.BlockSpec(memory_space=sm),
            pl.BlockSpec(memory_space=sm),
            pl.BlockSpec(memory_space=vm),
            pl.BlockSpec(memory_space=vm),
            pl.BlockSpec(memory_space=vm),
            pl.BlockSpec(memory_space=vm),
            pl.BlockSpec(memory_space=vm),
            pl.BlockSpec(memory_space=vm),
            pl.BlockSpec(memory_space=vm),
            pl.BlockSpec(memory_space=vm),
            pl.BlockSpec(memory_space=hb),
            pl.BlockSpec(memory_space=hb),
        ],
        out_specs=[
            pl.BlockSpec(memory_space=hb),
            pl.BlockSpec(memory_space=hb),
            pl.BlockSpec(memory_space=vm),
            pl.BlockSpec(memory_space=vm),
            pl.BlockSpec(memory_space=vm),
            pl.BlockSpec(memory_space=vm),
            pl.BlockSpec(memory_space=vm),
            pl.BlockSpec(memory_space=vm),
            pl.BlockSpec(memory_space=vm),
            pl.BlockSpec(memory_space=vm),
            pl.BlockSpec(memory_space=vm),
            pl.BlockSpec(memory_space=vm),
            pl.BlockSpec(memory_space=vm),
            pl.BlockSpec(memory_space=vm),
        ],
        out_shape=[
            jax.ShapeDtypeStruct((BUFF, C, FH * FW), jnp.float32),
            jax.ShapeDtypeStruct((BUFF, C, FH * FW), jnp.float32),
            jax.ShapeDtypeStruct((C, FH * FW), jnp.float32),
            jax.ShapeDtypeStruct((1, BUFF), jnp.float32),
            jax.ShapeDtypeStruct((BUFF, 7), jnp.float32),
            jax.ShapeDtypeStruct((BUFF, 3, 2, 128), jnp.float32),
            jax.ShapeDtypeStruct((BUFF, P), jnp.int32),
            jax.ShapeDtypeStruct((BUFF, SLOT), jnp.int32),
            jax.ShapeDtypeStruct((BUFF, SLOT), jnp.int32),
            jax.ShapeDtypeStruct((BUFF, SLOT), jnp.float32),
            jax.ShapeDtypeStruct((BUFF, SLOT), jnp.int8),
            jax.ShapeDtypeStruct((2, 128), jnp.int32),
            jax.ShapeDtypeStruct((2, 128), jnp.int32),
            jax.ShapeDtypeStruct((2, 128), jnp.int32),
        ],
        scratch_shapes=[
            pltpu.VMEM((C, FH * FW), jnp.float32),
            pltpu.SemaphoreType.DMA,
            pltpu.SemaphoreType.DMA,
        ],
        input_output_aliases={13: 0, 14: 1},
    )(xbT, sc2, wcat, fn1, time_stamp, time2, poses_buf, pstate_t,
      source_frame, i2, j2, w2, v2, fmap_hbm, imap_hbm)

    (fmap_o, imap_o, fmap_s, time_o, poses_o, pstate_o, sframe_o,
     i_o, j_o, w_o, v_o, idx_o, cy_o, cx_o) = outs

    fn_i = jnp.asarray(frame_n, jnp.int32)
    local = fn_i % BUFF
    cy256 = cy_o.reshape(P)
    cx256 = cx_o.reshape(P)
    new_patches = _sc_patches(fmap_s, cy256, cx256).reshape(P, C, PS, PS)
    patches_o = patches_buf.at[local].set(new_patches)

    return (fmap_o.reshape(BUFF, C, FH, FW),
            imap_o.reshape(BUFF, C, FH, FW),
            patches_o,
            pstate_o.reshape(BUFF, 3, P).transpose(0, 2, 1),
            poses_o,
            time_o.reshape(BUFF),
            sframe_o,
            i_o.reshape(MAX_EDGES),
            j_o.reshape(MAX_EDGES),
            w_o.reshape(MAX_EDGES),
            (v_o != 0).reshape(MAX_EDGES))
